# baseline (speedup 1.0000x reference)
import jax
import jax.numpy as jnp
from jax.experimental import pallas as pl

B, S, D = 1, 2048, 768
E = 10
EP = 16
TOP_K = 2
CITY_LEN = 10
CITY_EMB = 32
H = 768
FEAT = D + D // 4 + D // 4 + D // 8 + D // 8
BLK = 256
NBLK = S // BLK

_NEG = -1e30


def _router_kernel(feat_ref, wr_ref, wn_ref, coh_ref, cemb_ref, wce_r_ref,
                   wce_n_ref, br_ref, bn_ref, noise_ref,
                   gate1_ref, gating_ref):
    lane = jax.lax.broadcasted_iota(jnp.int32, (S, EP), 1)
    emask = lane < E

    ce_row = jnp.dot(coh_ref[...], cemb_ref[...],
                     preferred_element_type=jnp.float32)
    feat = feat_ref[...]
    logits = (jnp.dot(feat, wr_ref[...], preferred_element_type=jnp.float32)
              + jnp.dot(ce_row, wce_r_ref[...],
                        preferred_element_type=jnp.float32)
              + br_ref[...])
    nse = (jnp.dot(feat, wn_ref[...], preferred_element_type=jnp.float32)
           + jnp.dot(ce_row, wce_n_ref[...],
                     preferred_element_type=jnp.float32)
           + bn_ref[...])
    std = jnp.maximum(nse, 0.0) + jnp.log1p(jnp.exp(-jnp.abs(nse)))
    noisy = jnp.where(emask, logits + noise_ref[...] * std, _NEG)

    m1 = jnp.max(noisy, axis=1, keepdims=True)
    i1 = jnp.min(jnp.where(noisy == m1, lane, 999), axis=1, keepdims=True)
    noisy2 = jnp.where(lane == i1, _NEG, noisy)
    m2 = jnp.max(noisy2, axis=1, keepdims=True)
    i2 = jnp.min(jnp.where(noisy2 == m2, lane, 999), axis=1, keepdims=True)
    eb = jnp.exp(m2 - m1)
    g1 = 1.0 / (1.0 + eb)
    g2 = eb * g1
    gating_ref[...] = (jnp.where(lane == i1, g1, 0.0)
                       + jnp.where(lane == i2, g2, 0.0))

    lm = jnp.max(jnp.where(emask, logits, _NEG), axis=1, keepdims=True)
    ex = jnp.where(emask, jnp.exp(logits - lm), 0.0)
    gate1_ref[...] = ex / jnp.sum(ex, axis=1, keepdims=True)


def _expert_kernel(gating_ref, x_ref, w1_ref, b1_ref, w2_ref, b2_ref, out_ref):
    e = pl.program_id(0)
    j = pl.program_id(1)
    rows = pl.ds(j * BLK, BLK)

    h = jnp.maximum(
        jnp.dot(x_ref[...], w1_ref[0], preferred_element_type=jnp.float32)
        + b1_ref[0], 0.0)
    y = (jnp.dot(h, w2_ref[0], preferred_element_type=jnp.float32)
         + b2_ref[0])
    eoh = (jax.lax.broadcasted_iota(jnp.int32, (EP, 1), 0) == e
           ).astype(jnp.float32)
    g = jnp.dot(gating_ref[...], eoh, preferred_element_type=jnp.float32)
    contrib = y * g

    @pl.when(e == 0)
    def _():
        out_ref[rows, :] = contrib

    @pl.when(e != 0)
    def _():
        out_ref[rows, :] += contrib


def kernel(x, city, delta_t_info, delta_dis_info, delta_rg_info,
           delta_entropy_info, city_embeddings, route_W, route_b,
           noise_W, noise_b, W1, b1, W2, b2):
    x2d = x[0]
    feat = jnp.concatenate(
        [x2d, delta_t_info[0], delta_dis_info[0], delta_rg_info[0],
         delta_entropy_info[0]], axis=-1)

    def _padE(a):
        return jnp.pad(a, [(0, 0)] * (a.ndim - 1) + [(0, EP - E)])

    wr = _padE(jnp.concatenate([route_W[:, :D], route_W[:, D + CITY_EMB:]],
                               axis=1).T)
    wn = _padE(jnp.concatenate([noise_W[:, :D], noise_W[:, D + CITY_EMB:]],
                               axis=1).T)
    wce_r = _padE(route_W[:, D:D + CITY_EMB].T)
    wce_n = _padE(noise_W[:, D:D + CITY_EMB].T)
    br = _padE(route_b)[None, :]
    bn = _padE(noise_b)[None, :]
    coh = jax.nn.one_hot(city[0], EP, dtype=jnp.float32)[None, :]
    cemb = jnp.pad(city_embeddings, ((0, EP - CITY_LEN), (0, 0)))
    noise = _padE(jax.random.normal(jax.random.key(42), (S, E),
                                    dtype=jnp.float32))

    gate1, gating = pl.pallas_call(
        _router_kernel,
        out_shape=(jax.ShapeDtypeStruct((S, EP), jnp.float32),
                   jax.ShapeDtypeStruct((S, EP), jnp.float32)),
    )(feat, wr, wn, coh, cemb, wce_r, wce_n, br, bn, noise)

    out = pl.pallas_call(
        _expert_kernel,
        grid=(E, NBLK),
        in_specs=[
            pl.BlockSpec((BLK, EP), lambda e, j: (j, 0)),
            pl.BlockSpec((BLK, D), lambda e, j: (j, 0)),
            pl.BlockSpec((1, D, H), lambda e, j: (e, 0, 0)),
            pl.BlockSpec((1, 1, H), lambda e, j: (e, 0, 0)),
            pl.BlockSpec((1, H, D), lambda e, j: (e, 0, 0)),
            pl.BlockSpec((1, 1, D), lambda e, j: (e, 0, 0)),
        ],
        out_specs=pl.BlockSpec((S, D), lambda e, j: (0, 0)),
        out_shape=jax.ShapeDtypeStruct((S, D), jnp.float32),
    )(gating, x2d, W1, b1[:, None, :], W2, b2[:, None, :])

    return (out[None], gate1[:, :E][None])
